# baseline (speedup 1.0000x reference)
import functools

import jax
import jax.numpy as jnp
from jax import lax
from jax.experimental import pallas as pl
from jax.experimental.pallas import tpu as pltpu
from jax.experimental.pallas import tpu_sc as plsc

N = 50000
NWK = 16
CHUNK = 3136
TAIL = N - (NWK - 1) * CHUNK
NV = CHUNK // 16
NV_TAIL = TAIL // 16
NID = 1024
L = 16
COLS = NID // NWK
NST = 5 * L
NSTP = 128
ROW = NID + NWK * NSTP


def _loss_body(beta_hbm, pid_hbm, out_hbm,
               beta_v, pid_v, acc_v, red_v, seg_v, st2_v, stf_v, out_v,
               sem, sh_all):
    wid = lax.axis_index("s")
    base = wid * CHUNK

    cp_b = pltpu.make_async_copy(beta_hbm.at[pl.ds(base, CHUNK)], beta_v, sem)
    cp_p = pltpu.make_async_copy(pid_hbm.at[pl.ds(base, CHUNK)], pid_v, sem)
    cp_bt = pltpu.make_async_copy(beta_hbm.at[pl.ds(base, TAIL)],
                                  beta_v.at[pl.ds(0, TAIL)], sem)
    cp_pt = pltpu.make_async_copy(pid_hbm.at[pl.ds(base, TAIL)],
                                  pid_v.at[pl.ds(0, TAIL)], sem)

    @pl.when(wid < NWK - 1)
    def _():
        cp_b.start()
        cp_p.start()

    @pl.when(wid == NWK - 1)
    def _():
        cp_bt.start()
        cp_pt.start()

    neg1 = jnp.full((L,), -1.0, jnp.float32)

    def ibody(j, carry):
        acc_v[pl.ds(j * L, L)] = neg1
        return carry

    lax.fori_loop(0, L * NID // L, ibody, 0)

    @pl.when(wid < NWK - 1)
    def _():
        cp_b.wait()
        cp_p.wait()

    @pl.when(wid == NWK - 1)
    def _():
        cp_bt.wait()
        cp_pt.wait()

    pad_ids = jnp.full((L,), NID - 1, jnp.int32)

    def pbody(t, carry):
        pid_v[pl.ds(NV_TAIL * L + t * L, L)] = pad_ids
        return carry

    @pl.when(wid == NWK - 1)
    def _():
        lax.fori_loop(0, NV - NV_TAIL, pbody, 0)

    laneoff = lax.iota(jnp.int32, L) * NID

    def body(i, carry):
        s, c = carry
        for u in range(2):
            ids = pid_v[pl.ds(i * 2 * L + u * L, L)]
            bet = beta_v[pl.ds(i * 2 * L + u * L, L)]
            is0 = ids == 0
            s = s + jnp.where(is0, bet, 0.0)
            c = c + jnp.where(is0, 1.0, 0.0)
            idx = laneoff + ids
            g = plsc.load_gather(acc_v, [idx])
            plsc.store_scatter(acc_v, [idx], jnp.maximum(g, bet))
        return (s, c)

    zero = jnp.zeros((L,), jnp.float32)
    s, c = lax.fori_loop(0, NV // 2, body, (zero, zero))

    def rbody(j, carry):
        m = acc_v[pl.ds(j * L, L)]
        for k in range(1, L):
            m = jnp.maximum(m, acc_v[pl.ds(j * L + k * NID, L)])
        red_v[pl.ds(j * L, L)] = m
        return carry

    lax.fori_loop(0, NID // L, rbody, 0)

    def cbody(cb, carry):
        pltpu.async_copy(red_v.at[pl.ds(cb * COLS, COLS)],
                         sh_all.at[cb, pl.ds(wid * COLS, COLS)], sem)
        return carry

    lax.fori_loop(0, NWK, cbody, 0)
    row0 = wid * 0
    pltpu.make_async_copy(red_v, sh_all.at[row0, pl.ds(0, NID)], sem).wait()
    plsc.subcore_barrier()

    pltpu.sync_copy(sh_all.at[wid, pl.ds(0, NID)], seg_v)
    sig_v = jnp.zeros((L,), jnp.float32)
    pc_v = jnp.zeros((L,), jnp.float32)
    z_v = jnp.zeros((L,), jnp.float32)

    def jbody(j, carry):
        sig_v, pc_v, z_v = carry
        m = seg_v[pl.ds(j * L, L)]
        for k in range(1, NWK):
            m = jnp.maximum(m, seg_v[pl.ds(k * COLS + j * L, L)])
        gid = lax.iota(jnp.int32, L) + (wid * COLS + j * L)
        pres = (gid >= 1) & (gid < 1000) & (m >= 0.0)
        pc_v = pc_v + jnp.where(pres, 1.0, 0.0)
        sig_v = sig_v + jnp.where(pres, 1.0 - jnp.where(m > 0.0, m, 0.0), 0.0)
        z_v = z_v + jnp.where(pres & (m == 0.0), 1.0, 0.0)
        return (sig_v, pc_v, z_v)

    sig_v, pc_v, z_v = lax.fori_loop(0, COLS // L, jbody, (sig_v, pc_v, z_v))
    st2_v[pl.ds(0, L)] = sig_v
    st2_v[pl.ds(L, L)] = pc_v
    st2_v[pl.ds(2 * L, L)] = z_v
    st2_v[pl.ds(3 * L, L)] = s
    st2_v[pl.ds(4 * L, L)] = c
    pltpu.sync_copy(st2_v, sh_all.at[row0, pl.ds(NID + wid * NSTP, NSTP)])
    plsc.subcore_barrier()

    @pl.when(wid == 0)
    def _():
        pltpu.sync_copy(sh_all.at[row0, pl.ds(NID, NWK * NSTP)], stf_v)

        def fbody(k, carry):
            sig_a, pc_a, z_a, bgs_a, bgc_a = carry
            sig_a = sig_a + stf_v[pl.ds(k * NSTP, L)]
            pc_a = pc_a + stf_v[pl.ds(k * NSTP + L, L)]
            z_a = z_a + stf_v[pl.ds(k * NSTP + 2 * L, L)]
            bgs_a = bgs_a + stf_v[pl.ds(k * NSTP + 3 * L, L)]
            bgc_a = bgc_a + stf_v[pl.ds(k * NSTP + 4 * L, L)]
            return (sig_a, pc_a, z_a, bgs_a, bgc_a)

        sig_a, pc_a, z_a, bgs_a, bgc_a = lax.fori_loop(
            0, NWK, fbody, (zero, zero, zero, zero, zero))
        b0 = beta_v[pl.ds(0, L)][0]
        ones = jnp.ones((L,), jnp.float32)
        v_sig = ones * jnp.sum(sig_a) - (ones * jnp.sum(z_a)) * (ones * b0)
        v_pc = ones * jnp.sum(pc_a)
        v_bgs = ones * jnp.sum(bgs_a)
        v_bgc = ones * jnp.sum(bgc_a)
        v_out = v_sig / v_pc + 0.1 * (v_bgs / jnp.maximum(v_bgc, 1.0))
        out_v[...] = jnp.where(v_bgc > 0.0, v_out, 0.0)
        pltpu.sync_copy(out_v, out_hbm)


_loss = functools.partial(
    pl.kernel,
    out_type=jax.ShapeDtypeStruct((L,), jnp.float32),
    mesh=plsc.VectorSubcoreMesh(
        core_axis_name="c", subcore_axis_name="s",
        num_cores=1, num_subcores=NWK,
    ),
    compiler_params=pltpu.CompilerParams(needs_layout_passes=False),
    scratch_types=[
        pltpu.VMEM((CHUNK,), jnp.float32),
        pltpu.VMEM((CHUNK,), jnp.int32),
        pltpu.VMEM((L * NID,), jnp.float32),
        pltpu.VMEM((NID,), jnp.float32),
        pltpu.VMEM((NID,), jnp.float32),
        pltpu.VMEM((NSTP,), jnp.float32),
        pltpu.VMEM((NWK * NSTP,), jnp.float32),
        pltpu.VMEM((L,), jnp.float32),
        pltpu.SemaphoreType.DMA,
        pltpu.VMEM_SHARED((NWK, ROW), jnp.float32),
    ],
)(_loss_body)


def kernel(w, beta, x, y, particle_id):
    del w, x, y
    out = _loss(beta, particle_id)
    return out[0]


# speedup vs baseline: 1.1064x; 1.1064x over previous
import functools

import jax
import jax.numpy as jnp
from jax import lax
from jax.experimental import pallas as pl
from jax.experimental.pallas import tpu as pltpu
from jax.experimental.pallas import tpu_sc as plsc

N = 50000
NWK = 16
CHUNK = 3136
TAIL = N - (NWK - 1) * CHUNK
NV = CHUNK // 16
NV_TAIL = TAIL // 16
NID = 1024
L = 16
COLS = NID // NWK
NST = 5 * L
NSTP = 128
ROW = NID + NWK * NSTP


def _loss_body(beta_hbm, pid_hbm, init_hbm, out_hbm,
               beta_v, pid_v, acc_v, red_v, seg_v, st2_v, stf_v, out_v,
               sem, sh_all):
    wid = lax.axis_index("s")
    base = wid * CHUNK

    cp_b = pltpu.make_async_copy(beta_hbm.at[pl.ds(base, CHUNK)], beta_v, sem)
    cp_p = pltpu.make_async_copy(pid_hbm.at[pl.ds(base, CHUNK)], pid_v, sem)
    cp_bt = pltpu.make_async_copy(beta_hbm.at[pl.ds(base, TAIL)],
                                  beta_v.at[pl.ds(0, TAIL)], sem)
    cp_pt = pltpu.make_async_copy(pid_hbm.at[pl.ds(base, TAIL)],
                                  pid_v.at[pl.ds(0, TAIL)], sem)

    @pl.when(wid < NWK - 1)
    def _():
        cp_b.start()
        cp_p.start()

    @pl.when(wid == NWK - 1)
    def _():
        cp_bt.start()
        cp_pt.start()

    cp_init = pltpu.make_async_copy(init_hbm, acc_v, sem)
    cp_init.start()

    @pl.when(wid < NWK - 1)
    def _():
        cp_b.wait()
        cp_p.wait()

    @pl.when(wid == NWK - 1)
    def _():
        cp_bt.wait()
        cp_pt.wait()

    pad_ids = jnp.full((L,), NID - 1, jnp.int32)

    def pbody(t, carry):
        pid_v[pl.ds(NV_TAIL * L + t * L, L)] = pad_ids
        return carry

    @pl.when(wid == NWK - 1)
    def _():
        lax.fori_loop(0, NV - NV_TAIL, pbody, 0)

    cp_init.wait()

    laneoff = lax.iota(jnp.int32, L) * NID

    def body(i, carry):
        s, c = carry
        for u in range(2):
            ids = pid_v[pl.ds(i * 2 * L + u * L, L)]
            bet = beta_v[pl.ds(i * 2 * L + u * L, L)]
            is0 = ids == 0
            s = s + jnp.where(is0, bet, 0.0)
            c = c + jnp.where(is0, 1.0, 0.0)
            idx = laneoff + ids
            g = plsc.load_gather(acc_v, [idx])
            plsc.store_scatter(acc_v, [idx], jnp.maximum(g, bet))
        return (s, c)

    zero = jnp.zeros((L,), jnp.float32)
    s, c = lax.fori_loop(0, NV // 2, body, (zero, zero))

    def rbody(j, carry):
        m = acc_v[pl.ds(j * L, L)]
        for k in range(1, L):
            m = jnp.maximum(m, acc_v[pl.ds(j * L + k * NID, L)])
        red_v[pl.ds(j * L, L)] = m
        return carry

    lax.fori_loop(0, NID // L, rbody, 0)

    def cbody(cb, carry):
        pltpu.async_copy(red_v.at[pl.ds(cb * COLS, COLS)],
                         sh_all.at[cb, pl.ds(wid * COLS, COLS)], sem)
        return carry

    lax.fori_loop(0, NWK, cbody, 0)
    row0 = wid * 0
    pltpu.make_async_copy(red_v, sh_all.at[row0, pl.ds(0, NID)], sem).wait()
    plsc.subcore_barrier()

    pltpu.sync_copy(sh_all.at[wid, pl.ds(0, NID)], seg_v)
    sig_v = jnp.zeros((L,), jnp.float32)
    pc_v = jnp.zeros((L,), jnp.float32)
    z_v = jnp.zeros((L,), jnp.float32)

    def jbody(j, carry):
        sig_v, pc_v, z_v = carry
        m = seg_v[pl.ds(j * L, L)]
        for k in range(1, NWK):
            m = jnp.maximum(m, seg_v[pl.ds(k * COLS + j * L, L)])
        gid = lax.iota(jnp.int32, L) + (wid * COLS + j * L)
        pres = (gid >= 1) & (gid < 1000) & (m >= 0.0)
        pc_v = pc_v + jnp.where(pres, 1.0, 0.0)
        sig_v = sig_v + jnp.where(pres, 1.0 - jnp.where(m > 0.0, m, 0.0), 0.0)
        z_v = z_v + jnp.where(pres & (m == 0.0), 1.0, 0.0)
        return (sig_v, pc_v, z_v)

    sig_v, pc_v, z_v = lax.fori_loop(0, COLS // L, jbody, (sig_v, pc_v, z_v))
    st2_v[pl.ds(0, L)] = sig_v
    st2_v[pl.ds(L, L)] = pc_v
    st2_v[pl.ds(2 * L, L)] = z_v
    st2_v[pl.ds(3 * L, L)] = s
    st2_v[pl.ds(4 * L, L)] = c
    pltpu.sync_copy(st2_v, sh_all.at[row0, pl.ds(NID + wid * NSTP, NSTP)])
    plsc.subcore_barrier()

    @pl.when(wid == 0)
    def _():
        pltpu.sync_copy(sh_all.at[row0, pl.ds(NID, NWK * NSTP)], stf_v)

        def fbody(k, carry):
            sig_a, pc_a, z_a, bgs_a, bgc_a = carry
            sig_a = sig_a + stf_v[pl.ds(k * NSTP, L)]
            pc_a = pc_a + stf_v[pl.ds(k * NSTP + L, L)]
            z_a = z_a + stf_v[pl.ds(k * NSTP + 2 * L, L)]
            bgs_a = bgs_a + stf_v[pl.ds(k * NSTP + 3 * L, L)]
            bgc_a = bgc_a + stf_v[pl.ds(k * NSTP + 4 * L, L)]
            return (sig_a, pc_a, z_a, bgs_a, bgc_a)

        sig_a, pc_a, z_a, bgs_a, bgc_a = lax.fori_loop(
            0, NWK, fbody, (zero, zero, zero, zero, zero))
        b0 = beta_v[pl.ds(0, L)][0]
        ones = jnp.ones((L,), jnp.float32)
        v_sig = ones * jnp.sum(sig_a) - (ones * jnp.sum(z_a)) * (ones * b0)
        v_pc = ones * jnp.sum(pc_a)
        v_bgs = ones * jnp.sum(bgs_a)
        v_bgc = ones * jnp.sum(bgc_a)
        v_out = v_sig / v_pc + 0.1 * (v_bgs / jnp.maximum(v_bgc, 1.0))
        out_v[...] = jnp.where(v_bgc > 0.0, v_out, 0.0)
        pltpu.sync_copy(out_v, out_hbm)


_loss = functools.partial(
    pl.kernel,
    out_type=jax.ShapeDtypeStruct((L,), jnp.float32),
    mesh=plsc.VectorSubcoreMesh(
        core_axis_name="c", subcore_axis_name="s",
        num_cores=1, num_subcores=NWK,
    ),
    compiler_params=pltpu.CompilerParams(needs_layout_passes=False),
    scratch_types=[
        pltpu.VMEM((CHUNK,), jnp.float32),
        pltpu.VMEM((CHUNK,), jnp.int32),
        pltpu.VMEM((L * NID,), jnp.float32),
        pltpu.VMEM((NID,), jnp.float32),
        pltpu.VMEM((NID,), jnp.float32),
        pltpu.VMEM((NSTP,), jnp.float32),
        pltpu.VMEM((NWK * NSTP,), jnp.float32),
        pltpu.VMEM((L,), jnp.float32),
        pltpu.SemaphoreType.DMA,
        pltpu.VMEM_SHARED((NWK, ROW), jnp.float32),
    ],
)(_loss_body)


def kernel(w, beta, x, y, particle_id):
    del w, x, y
    init = jnp.full((L * NID,), -1.0, jnp.float32)
    out = _loss(beta, particle_id, init)
    return out[0]
